# baseline (speedup 1.0000x reference)
import functools

import jax
import jax.numpy as jnp
from jax import lax
from jax.experimental import pallas as pl
from jax.experimental.pallas import tpu as pltpu
from jax.experimental.pallas import tpu_sc as plsc

C = 80
H = W = 7
NPIX = H * W
P_OVER = 588
P_PAD = 592
NCH = P_PAD // 16
NVTOT = 40
K_UNC = 147
N_RAND = 49
P_FIN = K_UNC + N_RAND
SEL = 160
SELV = SEL // 16
RB = 8
RPW = 32
GSTRIDE = 64
INF = float("inf")


def _lex_lt(ka, ia, kb, ib):
    return (ka < kb) | ((ka == kb) & (ia < ib))


def _ce(a, b):
    sw = _lex_lt(b[0], b[1], a[0], a[1])
    lo = (jnp.where(sw, b[0], a[0]), jnp.where(sw, b[1], a[1]))
    hi = (jnp.where(sw, a[0], b[0]), jnp.where(sw, a[1], b[1]))
    return lo, hi


def _rev1(t):
    return (lax.rev(t[0], (0,)), lax.rev(t[1], (0,)))


def _vsort1(t):
    return plsc.sort_key_val(t[0], t[1])


def _ce_opt(a, b):
    if b is None:
        return a, None
    if a is None:
        return b, None
    return _ce(a, b)


def _bmerge(vl):
    n = len(vl)
    assert n & (n - 1) == 0
    if n == 1:
        return [None] if vl[0] is None else [_vsort1(vl[0])]
    d = n // 2
    vl = list(vl)
    for j in range(d):
        vl[j], vl[j + d] = _ce_opt(vl[j], vl[j + d])
    return _bmerge(vl[:d]) + _bmerge(vl[d:])


def _merge(a, b):
    n = len(a) + len(b)
    np2 = 1 << (n - 1).bit_length()
    vl = a + [None] * (np2 - n) + [_rev1(t) for t in reversed(b)]
    return _bmerge(vl)[:n]


def _bsort(vl):
    runs = [[_vsort1(t)] for t in vl]
    while len(runs) > 1:
        nxt = []
        for j in range(0, len(runs) - 1, 2):
            nxt.append(_merge(runs[j], runs[j + 1]))
        if len(runs) % 2:
            nxt.append(runs[-1])
        runs = nxt
    return runs[0]


def _sc_body(cls_hbm, flat_hbm, xy_hbm, selx_hbm, sely_hbm,
             cls_v, gidx_v, gt_v, xy_v, kbuf, ibuf, outx_v, outy_v, sem):
    i32 = jnp.int32
    f32 = jnp.float32
    wid = lax.axis_index("s") * 2 + lax.axis_index("c")
    base = wid * RPW
    iota = lax.iota(i32, 16)

    pltpu.sync_copy(cls_hbm.at[pl.ds(base, RPW)], cls_v)
    pltpu.sync_copy(xy_hbm.at[pl.ds(base * 2 * P_OVER, RPW * 2 * P_OVER)], xy_v)

    for h in range(2):
        cls_ch = cls_v[pl.ds(h * 16, 16)]
        plane0 = ((base + h * 16 + iota) * C + cls_ch) * NPIX
        rows = (h * 16 + iota) * GSTRIDE
        for k in range(GSTRIDE):
            plsc.store_scatter(gidx_v, [rows + k], plane0 + min(k, NPIX - 1))

    cps = []
    for r in range(RPW):
        cps.append(pltpu.async_copy(
            flat_hbm.at[gidx_v.at[pl.ds(r * GSTRIDE, GSTRIDE)]],
            gt_v.at[pl.ds(r * GSTRIDE, GSTRIDE)], sem))
    for cp in cps:
        cp.wait()

    def row_step(r, _):
        def chunk_step(c, _):
            start = jnp.minimum(c * 16, P_OVER - 16)
            pidx = start + iota
            fo = (r * P_OVER + pidx) * 2
            xr = plsc.load_gather(xy_v, [fo])
            yr = plsc.load_gather(xy_v, [fo + 1])
            x = xr * 7.0 - 0.5
            y = yr * 7.0 - 0.5
            x0i = x.astype(i32) - jnp.where(x < 0.0, 1, 0)
            y0i = y.astype(i32) - jnp.where(y < 0.0, 1, 0)
            x0 = x0i.astype(f32)
            y0 = y0i.astype(f32)
            x1 = x0 + 1.0
            y1 = y0 + 1.0
            wx1 = x - x0
            wx0 = 1.0 - wx1
            wy1 = y - y0
            wy0 = 1.0 - wy1
            gb = r * GSTRIDE

            def corner(xf, yf):
                valid = jnp.where(
                    (xf >= 0) & (xf <= W - 1) & (yf >= 0) & (yf <= H - 1),
                    jnp.float32(1.0), jnp.float32(0.0))
                xc = jnp.clip(xf, 0.0, W - 1.0).astype(i32)
                yc = jnp.clip(yf, 0.0, H - 1.0).astype(i32)
                g = plsc.load_gather(gt_v, [gb + yc * W + xc])
                return g * valid

            v00 = corner(x0, y0)
            v10 = corner(x1, y0)
            v01 = corner(x0, y1)
            v11 = corner(x1, y1)
            val = (v00 * (wx0 * wy0) + v10 * (wx1 * wy0)
                   + v01 * (wx0 * wy1) + v11 * (wx1 * wy1))
            dup = pidx < c * 16
            key = jnp.where(dup, INF, jnp.abs(val))
            kbuf[pl.ds(c * 16, 16)] = key
            ibuf[pl.ds(c * 16, 16)] = pidx
            return 0

        lax.fori_loop(0, NCH, chunk_step, 0, unroll=False)
        for c in range(NCH, NVTOT):
            kbuf[pl.ds(c * 16, 16)] = jnp.full((16,), INF, f32)
            ibuf[pl.ds(c * 16, 16)] = c * 16 + iota

        best = None
        for blk in range(4):
            vl = [(kbuf[pl.ds((blk * SELV + v) * 16, 16)],
                   ibuf[pl.ds((blk * SELV + v) * 16, 16)])
                  for v in range(SELV)]
            srt = _bsort(vl)
            if best is None:
                best = srt
            else:
                best = _merge(best, srt)[:SELV]

        for v in range(SELV):
            kbuf[pl.ds(v * 16, 16)] = best[v][0]
            ibuf[pl.ds(v * 16, 16)] = best[v][1]

        for p in range(4):
            off = p % 2
            for v in range(SELV):
                gi = v * 16 + iota
                par = jnp.clip(gi + jnp.where((gi & 1) == off, 1, -1), 0, SEL - 1)
                k0 = kbuf[pl.ds(v * 16, 16)]
                i0 = ibuf[pl.ds(v * 16, 16)]
                pk = plsc.load_gather(kbuf, [par])
                pi = plsc.load_gather(ibuf, [par])
                low_side = par > gi
                take = jnp.where(low_side, _lex_lt(pk, pi, k0, i0),
                                 _lex_lt(k0, i0, pk, pi))
                kbuf[pl.ds(v * 16, 16)] = jnp.where(take, pk, k0)
                ibuf[pl.ds(v * 16, 16)] = jnp.where(take, pi, i0)

        for v in range(SELV):
            iv = ibuf[pl.ds(v * 16, 16)]
            fo = (r * P_OVER + iv) * 2
            outx_v[pl.ds(v * 16, 16)] = plsc.load_gather(xy_v, [fo])
            outy_v[pl.ds(v * 16, 16)] = plsc.load_gather(xy_v, [fo + 1])
        pltpu.sync_copy(outx_v, selx_hbm.at[base + r])
        pltpu.sync_copy(outy_v, sely_hbm.at[base + r])
        return 0

    lax.fori_loop(0, RPW, row_step, 0, unroll=False)


_sc_topk = functools.partial(
    pl.kernel,
    compiler_params=pltpu.CompilerParams(needs_layout_passes=False),
    out_type=[
        jax.ShapeDtypeStruct((1024, SEL), jnp.float32),
        jax.ShapeDtypeStruct((1024, SEL), jnp.float32),
    ],
    mesh=plsc.VectorSubcoreMesh(core_axis_name="c", subcore_axis_name="s"),
    scratch_types=[
        pltpu.VMEM((RPW,), jnp.int32),
        pltpu.VMEM((RPW * GSTRIDE,), jnp.int32),
        pltpu.VMEM((RPW * GSTRIDE,), jnp.float32),
        pltpu.VMEM((RPW * 2 * P_OVER,), jnp.float32),
        pltpu.VMEM((NVTOT * 16,), jnp.float32),
        pltpu.VMEM((NVTOT * 16,), jnp.int32),
        pltpu.VMEM((SEL,), jnp.float32),
        pltpu.VMEM((SEL,), jnp.float32),
        pltpu.SemaphoreType.DMA,
    ],
)(_sc_body)


def _tc_body(img_ref, selx_ref, sely_ref, crand_ref, out_pl, out_fc):
    f32 = jnp.float32
    img = img_ref[...]
    sxT = jnp.transpose(selx_ref[...])
    syT = jnp.transpose(sely_ref[...])
    crand = crand_ref[...]

    ji = lax.broadcasted_iota(jnp.int32, (1, NPIX), 1)
    jx = (ji % W).astype(f32)
    iy = (ji // W).astype(f32)

    for r in range(RB):
        xs = jnp.concatenate([sxT[:K_UNC, r : r + 1], crand[r, :, 0:1]], axis=0)
        ys = jnp.concatenate([syT[:K_UNC, r : r + 1], crand[r, :, 1:2]], axis=0)
        out_fc[r, :, 0:1] = xs
        out_fc[r, :, 1:2] = ys
        xsc = xs * 7.0 - 0.5
        ysc = ys * 7.0 - 0.5
        tx = jnp.maximum(0.0, 1.0 - jnp.abs(xsc - jx))
        ty = jnp.maximum(0.0, 1.0 - jnp.abs(ysc - iy))
        A = tx * ty
        out_pl[r] = lax.dot_general(
            img[r], A, (((1,), (1,)), ((), ())), preferred_element_type=f32
        )


@jax.jit
def kernel(mask_coarse_logits, classes, coords_oversample, coords_random):
    R = mask_coarse_logits.shape[0]
    img = mask_coarse_logits.reshape(R, C, NPIX)
    flat = mask_coarse_logits.reshape(-1)
    xy = coords_oversample.reshape(-1)

    selx, sely = _sc_topk(classes, flat, xy)

    grid = R // RB
    point_logits, final_coords = pl.pallas_call(
        _tc_body,
        grid=(grid,),
        in_specs=[
            pl.BlockSpec((RB, C, NPIX), lambda i: (i, 0, 0)),
            pl.BlockSpec((RB, SEL), lambda i: (i, 0)),
            pl.BlockSpec((RB, SEL), lambda i: (i, 0)),
            pl.BlockSpec((RB, N_RAND, 2), lambda i: (i, 0, 0)),
        ],
        out_specs=[
            pl.BlockSpec((RB, C, P_FIN), lambda i: (i, 0, 0)),
            pl.BlockSpec((RB, P_FIN, 2), lambda i: (i, 0, 0)),
        ],
        out_shape=[
            jax.ShapeDtypeStruct((R, C, P_FIN), jnp.float32),
            jax.ShapeDtypeStruct((R, P_FIN, 2), jnp.float32),
        ],
    )(img, selx, sely, coords_random)
    return point_logits, final_coords


# speedup vs baseline: 1.0620x; 1.0620x over previous
import functools

import jax
import jax.numpy as jnp
from jax import lax
from jax.experimental import pallas as pl
from jax.experimental.pallas import tpu as pltpu
from jax.experimental.pallas import tpu_sc as plsc

C = 80
H = W = 7
NPIX = H * W
P_OVER = 588
P_PAD = 592
NCH = P_PAD // 16
NVTOT = 40
K_UNC = 147
N_RAND = 49
P_FIN = K_UNC + N_RAND
SEL = 160
SELV = SEL // 16
RB = 8
RPW = 32
GSTRIDE = 64
INF = float("inf")


def _lex_lt(ka, ia, kb, ib):
    return (ka < kb) | ((ka == kb) & (ia < ib))


def _ce(a, b):
    sw = _lex_lt(b[0], b[1], a[0], a[1])
    lo = (jnp.where(sw, b[0], a[0]), jnp.where(sw, b[1], a[1]))
    hi = (jnp.where(sw, a[0], b[0]), jnp.where(sw, a[1], b[1]))
    return lo, hi


def _rev1(t):
    return (lax.rev(t[0], (0,)), lax.rev(t[1], (0,)))


def _vsort1(t):
    return plsc.sort_key_val(t[0], t[1])


def _ce_opt(a, b):
    if b is None:
        return a, None
    if a is None:
        return b, None
    return _ce(a, b)


def _bmerge(vl):
    n = len(vl)
    assert n & (n - 1) == 0
    if n == 1:
        return [None] if vl[0] is None else [_vsort1(vl[0])]
    d = n // 2
    vl = list(vl)
    for j in range(d):
        vl[j], vl[j + d] = _ce_opt(vl[j], vl[j + d])
    return _bmerge(vl[:d]) + _bmerge(vl[d:])


def _merge(a, b):
    n = len(a) + len(b)
    np2 = 1 << (n - 1).bit_length()
    vl = a + [None] * (np2 - n) + [_rev1(t) for t in reversed(b)]
    return _bmerge(vl)[:n]


def _bsort(vl):
    runs = [[_vsort1(t)] for t in vl]
    while len(runs) > 1:
        nxt = []
        for j in range(0, len(runs) - 1, 2):
            nxt.append(_merge(runs[j], runs[j + 1]))
        if len(runs) % 2:
            nxt.append(runs[-1])
        runs = nxt
    return runs[0]


def _sc_body(gtl_hbm, xy_hbm, selx_hbm, sely_hbm,
             gt_v, xy_v, kbuf, ibuf, outx_v, outy_v, sem):
    i32 = jnp.int32
    f32 = jnp.float32
    wid = lax.axis_index("s") * 2 + lax.axis_index("c")
    base = wid * RPW
    iota = lax.iota(i32, 16)

    pltpu.sync_copy(gtl_hbm.at[pl.ds(base * GSTRIDE, RPW * GSTRIDE)], gt_v)
    pltpu.sync_copy(xy_hbm.at[pl.ds(base * 2 * P_OVER, RPW * 2 * P_OVER)], xy_v)

    def row_step(r, _):
        def chunk_step(c, _):
            start = jnp.minimum(c * 16, P_OVER - 16)
            pidx = start + iota
            fo = (r * P_OVER + pidx) * 2
            xr = plsc.load_gather(xy_v, [fo])
            yr = plsc.load_gather(xy_v, [fo + 1])
            x = xr * 7.0 - 0.5
            y = yr * 7.0 - 0.5
            x0i = x.astype(i32) - jnp.where(x < 0.0, 1, 0)
            y0i = y.astype(i32) - jnp.where(y < 0.0, 1, 0)
            x0 = x0i.astype(f32)
            y0 = y0i.astype(f32)
            x1 = x0 + 1.0
            y1 = y0 + 1.0
            wx1 = x - x0
            wx0 = 1.0 - wx1
            wy1 = y - y0
            wy0 = 1.0 - wy1
            gb = r * GSTRIDE

            def corner(xf, yf):
                valid = jnp.where(
                    (xf >= 0) & (xf <= W - 1) & (yf >= 0) & (yf <= H - 1),
                    jnp.float32(1.0), jnp.float32(0.0))
                xc = jnp.clip(xf, 0.0, W - 1.0).astype(i32)
                yc = jnp.clip(yf, 0.0, H - 1.0).astype(i32)
                g = plsc.load_gather(gt_v, [gb + yc * W + xc])
                return g * valid

            v00 = corner(x0, y0)
            v10 = corner(x1, y0)
            v01 = corner(x0, y1)
            v11 = corner(x1, y1)
            val = (v00 * (wx0 * wy0) + v10 * (wx1 * wy0)
                   + v01 * (wx0 * wy1) + v11 * (wx1 * wy1))
            dup = pidx < c * 16
            key = jnp.where(dup, INF, jnp.abs(val))
            kbuf[pl.ds(c * 16, 16)] = key
            ibuf[pl.ds(c * 16, 16)] = pidx
            return 0

        lax.fori_loop(0, NCH, chunk_step, 0, unroll=False)
        for c in range(NCH, NVTOT):
            kbuf[pl.ds(c * 16, 16)] = jnp.full((16,), INF, f32)
            ibuf[pl.ds(c * 16, 16)] = c * 16 + iota

        best = None
        for blk in range(4):
            vl = [(kbuf[pl.ds((blk * SELV + v) * 16, 16)],
                   ibuf[pl.ds((blk * SELV + v) * 16, 16)])
                  for v in range(SELV)]
            srt = _bsort(vl)
            if best is None:
                best = srt
            else:
                best = _merge(best, srt)[:SELV]

        for v in range(SELV):
            kbuf[pl.ds(v * 16, 16)] = best[v][0]
            ibuf[pl.ds(v * 16, 16)] = best[v][1]

        for p in range(4):
            off = p % 2
            for v in range(SELV):
                gi = v * 16 + iota
                par = jnp.clip(gi + jnp.where((gi & 1) == off, 1, -1), 0, SEL - 1)
                k0 = kbuf[pl.ds(v * 16, 16)]
                i0 = ibuf[pl.ds(v * 16, 16)]
                pk = plsc.load_gather(kbuf, [par])
                pi = plsc.load_gather(ibuf, [par])
                low_side = par > gi
                take = jnp.where(low_side, _lex_lt(pk, pi, k0, i0),
                                 _lex_lt(k0, i0, pk, pi))
                kbuf[pl.ds(v * 16, 16)] = jnp.where(take, pk, k0)
                ibuf[pl.ds(v * 16, 16)] = jnp.where(take, pi, i0)

        for v in range(SELV):
            iv = ibuf[pl.ds(v * 16, 16)]
            fo = (r * P_OVER + iv) * 2
            outx_v[pl.ds(v * 16, 16)] = plsc.load_gather(xy_v, [fo])
            outy_v[pl.ds(v * 16, 16)] = plsc.load_gather(xy_v, [fo + 1])
        pltpu.sync_copy(outx_v, selx_hbm.at[base + r])
        pltpu.sync_copy(outy_v, sely_hbm.at[base + r])
        return 0

    lax.fori_loop(0, RPW, row_step, 0, unroll=False)


_sc_topk = functools.partial(
    pl.kernel,
    compiler_params=pltpu.CompilerParams(needs_layout_passes=False),
    out_type=[
        jax.ShapeDtypeStruct((1024, SEL), jnp.float32),
        jax.ShapeDtypeStruct((1024, SEL), jnp.float32),
    ],
    mesh=plsc.VectorSubcoreMesh(core_axis_name="c", subcore_axis_name="s"),
    scratch_types=[
        pltpu.VMEM((RPW * GSTRIDE,), jnp.float32),
        pltpu.VMEM((RPW * 2 * P_OVER,), jnp.float32),
        pltpu.VMEM((NVTOT * 16,), jnp.float32),
        pltpu.VMEM((NVTOT * 16,), jnp.int32),
        pltpu.VMEM((SEL,), jnp.float32),
        pltpu.VMEM((SEL,), jnp.float32),
        pltpu.SemaphoreType.DMA,
    ],
)(_sc_body)


def _tc_gt_body(cls_ref, img_ref, out_gt):
    cls = cls_ref[...]
    img = img_ref[...]
    ch_iota = lax.broadcasted_iota(jnp.int32, (1, C), 1)
    oh = cls == ch_iota
    g = jnp.zeros((RB, NPIX), jnp.float32)
    for c in range(C):
        g = g + jnp.where(oh[:, c : c + 1], img[:, c, :], 0.0)
    out_gt[:, :NPIX] = g
    out_gt[:, NPIX:] = jnp.zeros((RB, GSTRIDE - NPIX), jnp.float32)


def _tc_body(img_ref, selx_ref, sely_ref, crand_ref, out_pl, out_fc):
    f32 = jnp.float32
    img = img_ref[...]
    sxT = jnp.transpose(selx_ref[...])
    syT = jnp.transpose(sely_ref[...])
    crand = crand_ref[...]

    ji = lax.broadcasted_iota(jnp.int32, (1, NPIX), 1)
    jx = (ji % W).astype(f32)
    iy = (ji // W).astype(f32)

    for r in range(RB):
        xs = jnp.concatenate([sxT[:K_UNC, r : r + 1], crand[r, :, 0:1]], axis=0)
        ys = jnp.concatenate([syT[:K_UNC, r : r + 1], crand[r, :, 1:2]], axis=0)
        out_fc[r, :, 0:1] = xs
        out_fc[r, :, 1:2] = ys
        xsc = xs * 7.0 - 0.5
        ysc = ys * 7.0 - 0.5
        tx = jnp.maximum(0.0, 1.0 - jnp.abs(xsc - jx))
        ty = jnp.maximum(0.0, 1.0 - jnp.abs(ysc - iy))
        A = tx * ty
        out_pl[r] = lax.dot_general(
            img[r], A, (((1,), (1,)), ((), ())), preferred_element_type=f32
        )


@jax.jit
def kernel(mask_coarse_logits, classes, coords_oversample, coords_random):
    R = mask_coarse_logits.shape[0]
    img = mask_coarse_logits.reshape(R, C, NPIX)
    xy = coords_oversample.reshape(-1)
    grid = R // RB

    gt64 = pl.pallas_call(
        _tc_gt_body,
        grid=(grid,),
        in_specs=[
            pl.BlockSpec((RB, 1), lambda i: (i, 0)),
            pl.BlockSpec((RB, C, NPIX), lambda i: (i, 0, 0)),
        ],
        out_specs=pl.BlockSpec((RB, GSTRIDE), lambda i: (i, 0)),
        out_shape=jax.ShapeDtypeStruct((R, GSTRIDE), jnp.float32),
    )(classes[:, None], img)

    selx, sely = _sc_topk(gt64.reshape(-1), xy)
    point_logits, final_coords = pl.pallas_call(
        _tc_body,
        grid=(grid,),
        in_specs=[
            pl.BlockSpec((RB, C, NPIX), lambda i: (i, 0, 0)),
            pl.BlockSpec((RB, SEL), lambda i: (i, 0)),
            pl.BlockSpec((RB, SEL), lambda i: (i, 0)),
            pl.BlockSpec((RB, N_RAND, 2), lambda i: (i, 0, 0)),
        ],
        out_specs=[
            pl.BlockSpec((RB, C, P_FIN), lambda i: (i, 0, 0)),
            pl.BlockSpec((RB, P_FIN, 2), lambda i: (i, 0, 0)),
        ],
        out_shape=[
            jax.ShapeDtypeStruct((R, C, P_FIN), jnp.float32),
            jax.ShapeDtypeStruct((R, P_FIN, 2), jnp.float32),
        ],
    )(img, selx, sely, coords_random)
    return point_logits, final_coords


# speedup vs baseline: 1.6484x; 1.5521x over previous
import functools

import jax
import jax.numpy as jnp
from jax import lax
from jax.experimental import pallas as pl
from jax.experimental.pallas import tpu as pltpu
from jax.experimental.pallas import tpu_sc as plsc

C = 80
H = W = 7
NPIX = H * W
P_OVER = 588
P_PAD = 592
NCH = P_PAD // 16
NVTOT = 40
K_UNC = 147
N_RAND = 49
P_FIN = K_UNC + N_RAND
SEL = 160
SELV = SEL // 16
RB = 8
RPW = 32
GSTRIDE = 64
INF = float("inf")


def _lex_lt(ka, ia, kb, ib):
    return (ka < kb) | ((ka == kb) & (ia < ib))


def _ce(a, b):
    sw = _lex_lt(b[0], b[1], a[0], a[1])
    lo = (jnp.where(sw, b[0], a[0]), jnp.where(sw, b[1], a[1]))
    hi = (jnp.where(sw, a[0], b[0]), jnp.where(sw, a[1], b[1]))
    return lo, hi


def _rev1(t):
    return (lax.rev(t[0], (0,)), lax.rev(t[1], (0,)))


def _vsort1(t):
    return plsc.sort_key_val(t[0], t[1])


def _ce_opt(a, b):
    if b is None:
        return a, None
    if a is None:
        return b, None
    return _ce(a, b)


def _bmerge(vl):
    n = len(vl)
    assert n & (n - 1) == 0
    if n == 1:
        return [None] if vl[0] is None else [_vsort1(vl[0])]
    d = n // 2
    vl = list(vl)
    for j in range(d):
        vl[j], vl[j + d] = _ce_opt(vl[j], vl[j + d])
    return _bmerge(vl[:d]) + _bmerge(vl[d:])


def _merge(a, b):
    n = len(a) + len(b)
    np2 = 1 << (n - 1).bit_length()
    vl = a + [None] * (np2 - n) + [_rev1(t) for t in reversed(b)]
    return _bmerge(vl)[:n]


def _bsort(vl):
    runs = [[_vsort1(t)] for t in vl]
    while len(runs) > 1:
        nxt = []
        for j in range(0, len(runs) - 1, 2):
            nxt.append(_merge(runs[j], runs[j + 1]))
        if len(runs) % 2:
            nxt.append(runs[-1])
        runs = nxt
    return runs[0]


def _sc_body(gtl_hbm, xo_hbm, yo_hbm, selx_hbm, sely_hbm,
             gt_v, xo_v, yo_v, kbuf, ibuf, outx_v, outy_v, sem):
    i32 = jnp.int32
    f32 = jnp.float32
    wid = lax.axis_index("s") * 2 + lax.axis_index("c")
    base = wid * RPW
    iota = lax.iota(i32, 16)

    pltpu.sync_copy(gtl_hbm.at[pl.ds(base * GSTRIDE, RPW * GSTRIDE)], gt_v)
    pltpu.sync_copy(xo_hbm.at[pl.ds(base * P_PAD, RPW * P_PAD)], xo_v)
    pltpu.sync_copy(yo_hbm.at[pl.ds(base * P_PAD, RPW * P_PAD)], yo_v)

    def row_step(r, _):
        def chunk_step(c, _):
            po = r * P_PAD + c * 16
            xr = xo_v[pl.ds(po, 16)]
            yr = yo_v[pl.ds(po, 16)]
            x = xr * 7.0 - 0.5
            y = yr * 7.0 - 0.5
            x0i = x.astype(i32) - jnp.where(x < 0.0, 1, 0)
            y0i = y.astype(i32) - jnp.where(y < 0.0, 1, 0)
            x0 = x0i.astype(f32)
            y0 = y0i.astype(f32)
            x1 = x0 + 1.0
            y1 = y0 + 1.0
            wx1 = x - x0
            wx0 = 1.0 - wx1
            wy1 = y - y0
            wy0 = 1.0 - wy1
            gb = r * GSTRIDE

            def corner(xf, yf):
                valid = jnp.where(
                    (xf >= 0) & (xf <= W - 1) & (yf >= 0) & (yf <= H - 1),
                    jnp.float32(1.0), jnp.float32(0.0))
                xc = jnp.clip(xf, 0.0, W - 1.0).astype(i32)
                yc = jnp.clip(yf, 0.0, H - 1.0).astype(i32)
                g = plsc.load_gather(gt_v, [gb + yc * W + xc])
                return g * valid

            v00 = corner(x0, y0)
            v10 = corner(x1, y0)
            v01 = corner(x0, y1)
            v11 = corner(x1, y1)
            val = (v00 * (wx0 * wy0) + v10 * (wx1 * wy0)
                   + v01 * (wx0 * wy1) + v11 * (wx1 * wy1))
            idx = c * 16 + iota
            key = jnp.where(idx < P_OVER, jnp.abs(val), INF)
            kbuf[pl.ds(c * 16, 16)] = key
            ibuf[pl.ds(c * 16, 16)] = idx
            return 0

        lax.fori_loop(0, NCH, chunk_step, 0, unroll=False)
        for c in range(NCH, NVTOT):
            kbuf[pl.ds(c * 16, 16)] = jnp.full((16,), INF, f32)
            ibuf[pl.ds(c * 16, 16)] = c * 16 + iota

        best = None
        for blk in range(4):
            vl = [(kbuf[pl.ds((blk * SELV + v) * 16, 16)],
                   ibuf[pl.ds((blk * SELV + v) * 16, 16)])
                  for v in range(SELV)]
            srt = _bsort(vl)
            if best is None:
                best = srt
            else:
                best = _merge(best, srt)[:SELV]

        for v in range(SELV):
            kbuf[pl.ds(v * 16, 16)] = best[v][0]
            ibuf[pl.ds(v * 16, 16)] = best[v][1]

        for p in range(4):
            off = p % 2
            for v in range(SELV):
                gi = v * 16 + iota
                par = jnp.clip(gi + jnp.where((gi & 1) == off, 1, -1), 0, SEL - 1)
                k0 = kbuf[pl.ds(v * 16, 16)]
                i0 = ibuf[pl.ds(v * 16, 16)]
                pk = plsc.load_gather(kbuf, [par])
                pi = plsc.load_gather(ibuf, [par])
                low_side = par > gi
                take = jnp.where(low_side, _lex_lt(pk, pi, k0, i0),
                                 _lex_lt(k0, i0, pk, pi))
                kbuf[pl.ds(v * 16, 16)] = jnp.where(take, pk, k0)
                ibuf[pl.ds(v * 16, 16)] = jnp.where(take, pi, i0)

        for v in range(SELV):
            iv = ibuf[pl.ds(v * 16, 16)]
            outx_v[pl.ds(v * 16, 16)] = plsc.load_gather(xo_v, [r * P_PAD + iv])
            outy_v[pl.ds(v * 16, 16)] = plsc.load_gather(yo_v, [r * P_PAD + iv])
        pltpu.sync_copy(outx_v, selx_hbm.at[base + r])
        pltpu.sync_copy(outy_v, sely_hbm.at[base + r])
        return 0

    lax.fori_loop(0, RPW, row_step, 0, unroll=False)


_sc_topk = functools.partial(
    pl.kernel,
    compiler_params=pltpu.CompilerParams(needs_layout_passes=False),
    out_type=[
        jax.ShapeDtypeStruct((1024, SEL), jnp.float32),
        jax.ShapeDtypeStruct((1024, SEL), jnp.float32),
    ],
    mesh=plsc.VectorSubcoreMesh(core_axis_name="c", subcore_axis_name="s"),
    scratch_types=[
        pltpu.VMEM((RPW * GSTRIDE,), jnp.float32),
        pltpu.VMEM((RPW * P_PAD,), jnp.float32),
        pltpu.VMEM((RPW * P_PAD,), jnp.float32),
        pltpu.VMEM((NVTOT * 16,), jnp.float32),
        pltpu.VMEM((NVTOT * 16,), jnp.int32),
        pltpu.VMEM((SEL,), jnp.float32),
        pltpu.VMEM((SEL,), jnp.float32),
        pltpu.SemaphoreType.DMA,
    ],
)(_sc_body)


def _tc_gt_body(cls_ref, img_ref, out_gt):
    cls = cls_ref[...]
    img = img_ref[...]
    ch_iota = lax.broadcasted_iota(jnp.int32, (1, C), 1)
    oh = cls == ch_iota
    g = jnp.zeros((RB, NPIX), jnp.float32)
    for c in range(C):
        g = g + jnp.where(oh[:, c : c + 1], img[:, c, :], 0.0)
    out_gt[:, :NPIX] = g
    out_gt[:, NPIX:] = jnp.zeros((RB, GSTRIDE - NPIX), jnp.float32)


def _tc_body(img_ref, selx_ref, sely_ref, crand_ref, out_pl, out_fc):
    f32 = jnp.float32
    img = img_ref[...]
    sxT = jnp.transpose(selx_ref[...])
    syT = jnp.transpose(sely_ref[...])
    crand = crand_ref[...]

    ji = lax.broadcasted_iota(jnp.int32, (1, NPIX), 1)
    jx = (ji % W).astype(f32)
    iy = (ji // W).astype(f32)

    for r in range(RB):
        xs = jnp.concatenate([sxT[:K_UNC, r : r + 1], crand[r, :, 0:1]], axis=0)
        ys = jnp.concatenate([syT[:K_UNC, r : r + 1], crand[r, :, 1:2]], axis=0)
        out_fc[r, :, 0:1] = xs
        out_fc[r, :, 1:2] = ys
        xsc = xs * 7.0 - 0.5
        ysc = ys * 7.0 - 0.5
        tx = jnp.maximum(0.0, 1.0 - jnp.abs(xsc - jx))
        ty = jnp.maximum(0.0, 1.0 - jnp.abs(ysc - iy))
        A = tx * ty
        out_pl[r] = lax.dot_general(
            img[r], A, (((1,), (1,)), ((), ())), preferred_element_type=f32
        )


@jax.jit
def kernel(mask_coarse_logits, classes, coords_oversample, coords_random):
    R = mask_coarse_logits.shape[0]
    img = mask_coarse_logits.reshape(R, C, NPIX)
    xo = jnp.pad(coords_oversample[..., 0], ((0, 0), (0, P_PAD - P_OVER))).reshape(-1)
    yo = jnp.pad(coords_oversample[..., 1], ((0, 0), (0, P_PAD - P_OVER))).reshape(-1)
    grid = R // RB

    gt64 = pl.pallas_call(
        _tc_gt_body,
        grid=(grid,),
        in_specs=[
            pl.BlockSpec((RB, 1), lambda i: (i, 0)),
            pl.BlockSpec((RB, C, NPIX), lambda i: (i, 0, 0)),
        ],
        out_specs=pl.BlockSpec((RB, GSTRIDE), lambda i: (i, 0)),
        out_shape=jax.ShapeDtypeStruct((R, GSTRIDE), jnp.float32),
    )(classes[:, None], img)

    selx, sely = _sc_topk(gt64.reshape(-1), xo, yo)
    point_logits, final_coords = pl.pallas_call(
        _tc_body,
        grid=(grid,),
        in_specs=[
            pl.BlockSpec((RB, C, NPIX), lambda i: (i, 0, 0)),
            pl.BlockSpec((RB, SEL), lambda i: (i, 0)),
            pl.BlockSpec((RB, SEL), lambda i: (i, 0)),
            pl.BlockSpec((RB, N_RAND, 2), lambda i: (i, 0, 0)),
        ],
        out_specs=[
            pl.BlockSpec((RB, C, P_FIN), lambda i: (i, 0, 0)),
            pl.BlockSpec((RB, P_FIN, 2), lambda i: (i, 0, 0)),
        ],
        out_shape=[
            jax.ShapeDtypeStruct((R, C, P_FIN), jnp.float32),
            jax.ShapeDtypeStruct((R, P_FIN, 2), jnp.float32),
        ],
    )(img, selx, sely, coords_random)
    return point_logits, final_coords
